# Initial kernel scaffold; baseline (speedup 1.0000x reference)
import jax
import jax.numpy as jnp
from jax.experimental import pallas as pl


def kernel(user_emb, item_emb, adj_values, adj_indices):
    raise NotImplementedError("write your pallas kernel here")


# speedup vs baseline: 3.4173x; 3.4173x over previous
import functools

import jax
import jax.numpy as jnp
from jax import lax
from jax.experimental import pallas as pl
from jax.experimental.pallas import tpu as pltpu
from jax.experimental.pallas import tpu_sc as plsc

USER_N = 25000
ITEM_N = 25000
NODES = USER_N + ITEM_N
EMB = 64
HALF = EMB // 2
LAYERS = 3
EDGES = 800000

NC = 2
NS = 16
LANES = 16
CHUNK = 128
NP = 50048
ROWS_PER_TILE = NP // NS
PIECE = 136
NPIECE = ROWS_PER_TILE // PIECE
NCHUNK = -(-EDGES // (NS * CHUNK))
EDGES_PAD = NS * NCHUNK * CHUNK


def _build_kernel():
    mesh = plsc.VectorSubcoreMesh(core_axis_name="c", subcore_axis_name="s")

    @functools.partial(
        pl.kernel,
        out_type=(
            jax.ShapeDtypeStruct((LAYERS * NC * NP, HALF), jnp.float32),
            jax.ShapeDtypeStruct((NC * NP, HALF), jnp.float32),
        ),
        mesh=mesh,
        compiler_params=pltpu.CompilerParams(use_tc_tiling_on_sc=False),
        scratch_types=[
            pltpu.VMEM((2, CHUNK), jnp.int32),
            pltpu.VMEM((1, CHUNK), jnp.float32),
            pltpu.VMEM((CHUNK, HALF), jnp.float32),
            pltpu.VMEM((PIECE, HALF), jnp.float32),
            pltpu.VMEM((PIECE, HALF), jnp.float32),
            pltpu.VMEM((PIECE, HALF), jnp.float32),
            pltpu.VMEM((PIECE, HALF), jnp.float32),
            pltpu.VMEM((PIECE, HALF), jnp.float32),
            pltpu.VMEM_SHARED((NP, HALF), jnp.float32),
            pltpu.SemaphoreType.DMA,
        ],
    )
    def lgcn_kernel(table0, blob, vals, t_out, mean_out,
                    ebuf, vbuf, rows, pbuf, p1, p2, p3, zbuf, acc, gsem):
        c = lax.axis_index("c")
        s = lax.axis_index("s")
        half_off = c * NP
        r0 = s * ROWS_PER_TILE

        zv = jnp.zeros((LANES,), jnp.float32)
        for r in range(PIECE):
            for h in range(HALF // LANES):
                zbuf[r, pl.ds(h * LANES, LANES)] = zv

        def edge_chunk(j, gather_base):
            pltpu.sync_copy(blob.at[s, j], ebuf)
            pltpu.sync_copy(vals.at[s, j], vbuf)
            for k in range(CHUNK // LANES):
                sl = pl.ds(k * LANES, LANES)
                ebuf[0, sl] = ebuf[0, sl] + gather_base
            return ebuf

        def mul_rows():
            for g in range(CHUNK // LANES):
                vv = vbuf[0, pl.ds(g * LANES, LANES)]
                for e in range(LANES):
                    b = jnp.take_along_axis(
                        vv, jnp.full((LANES,), e, jnp.int32), axis=0)
                    r = g * LANES + e
                    for h in range(HALF // LANES):
                        sl = pl.ds(h * LANES, LANES)
                        rows[r, sl] = rows[r, sl] * b

        def zero_acc():
            def zbody(p, _):
                pltpu.sync_copy(zbuf, acc.at[pl.ds(r0 + p * PIECE, PIECE)])
                return _
            lax.fori_loop(0, NPIECE, zbody, 0)

        def writeback(out_base):
            def wbody(p, _):
                off = r0 + p * PIECE
                pltpu.sync_copy(acc.at[pl.ds(off, PIECE)], pbuf)
                pltpu.sync_copy(
                    pbuf, t_out.at[pl.ds(out_base + half_off + off, PIECE)])
                return _
            lax.fori_loop(0, NPIECE, wbody, 0)

        def layer(table_ref, gather_base, out_base):
            zero_acc()
            plsc.subcore_barrier()

            def ebody(j, _):
                edge_chunk(j, gather_base)
                pltpu.async_copy(table_ref.at[ebuf.at[0]], rows, gsem).wait()
                mul_rows()
                pltpu.sync_copy(rows, acc.at[ebuf.at[1]], add=True)
                return _
            lax.fori_loop(0, NCHUNK, ebody, 0)
            plsc.subcore_barrier()
            writeback(out_base)
            plsc.subcore_barrier()

        layer(table0, half_off, 0)

        def lbody(i, _):
            layer(t_out, half_off + i * (NC * NP), (i + 1) * (NC * NP))
            return _
        lax.fori_loop(0, LAYERS - 1, lbody, 0)

        def mbody(p, _):
            hb = half_off + r0 + p * PIECE
            pltpu.sync_copy(table0.at[pl.ds(hb, PIECE)], pbuf)
            pltpu.sync_copy(t_out.at[pl.ds(hb, PIECE)], p1)
            pltpu.sync_copy(t_out.at[pl.ds(NC * NP + hb, PIECE)], p2)
            pltpu.sync_copy(t_out.at[pl.ds(2 * NC * NP + hb, PIECE)], p3)

            def rbody(r, _):
                for h in range(HALF // LANES):
                    sl = pl.ds(h * LANES, LANES)
                    m = (pbuf[r, sl] + p1[r, sl] + p2[r, sl] + p3[r, sl]) \
                        * 0.25
                    pbuf[r, sl] = m
                return _
            lax.fori_loop(0, PIECE, rbody, 0)
            pltpu.sync_copy(pbuf, mean_out.at[pl.ds(hb, PIECE)])
            return _
        lax.fori_loop(0, NPIECE, mbody, 0)

    return lgcn_kernel


_LGCN = _build_kernel()


def kernel(user_emb, item_emb, adj_values, adj_indices):
    emb0 = jnp.concatenate([user_emb, item_emb], axis=0)
    rpad = NP - NODES
    table0 = jnp.concatenate(
        [jnp.pad(emb0[:, :HALF], ((0, rpad), (0, 0))),
         jnp.pad(emb0[:, HALF:], ((0, rpad), (0, 0)))], axis=0)

    row = adj_indices[0]
    col = adj_indices[1]
    pad = EDGES_PAD - EDGES
    colp = jnp.pad(col, (0, pad))
    rowp = jnp.pad(row, (0, pad))
    valp = jnp.pad(adj_values, (0, pad))
    blob = jnp.stack([colp, rowp], axis=0)
    blob = blob.reshape(2, NS, NCHUNK, CHUNK).transpose(1, 2, 0, 3)
    vals = valp.reshape(NS, NCHUNK, 1, CHUNK)

    t_out, mean_out = _LGCN(table0, blob, vals)

    def unsplit(t):
        return jnp.concatenate([t[:NODES], t[NP:NP + NODES]], axis=1)

    layers = t_out.reshape(LAYERS, NC * NP, HALF)
    all_emb = jnp.stack([emb0] + [unsplit(layers[i]) for i in range(LAYERS)],
                        axis=0)
    final = unsplit(mean_out)
    return (final[:USER_N], final[USER_N:], all_emb)


# speedup vs baseline: 4.9958x; 1.4619x over previous
import functools

import jax
import jax.numpy as jnp
from jax import lax
from jax.experimental import pallas as pl
from jax.experimental.pallas import tpu as pltpu
from jax.experimental.pallas import tpu_sc as plsc

USER_N = 25000
ITEM_N = 25000
NODES = USER_N + ITEM_N
EMB = 64
HALF = EMB // 2
LAYERS = 3
EDGES = 800000

NC = 2
NS = 16
LANES = 16
CHUNK = 128
NP = 50048
ROWS_PER_TILE = NP // NS
PIECE = 136
NPIECE = ROWS_PER_TILE // PIECE
_NCH = -(-EDGES // (NS * CHUNK))
NCHUNK = _NCH + (_NCH % 2)
NCHUNK_ALLOC = NCHUNK + 2
EDGES_PAD = NS * NCHUNK_ALLOC * CHUNK
SLAB = NC * NP


def _build_kernel():
    mesh = plsc.VectorSubcoreMesh(core_axis_name="c", subcore_axis_name="s")

    @functools.partial(
        pl.kernel,
        out_type=(
            jax.ShapeDtypeStruct(((LAYERS + 1) * SLAB, HALF), jnp.float32),
            jax.ShapeDtypeStruct((SLAB, HALF), jnp.float32),
        ),
        mesh=mesh,
        compiler_params=pltpu.CompilerParams(use_tc_tiling_on_sc=False),
        scratch_types=[
            pltpu.VMEM((2, CHUNK), jnp.int32),
            pltpu.VMEM((2, CHUNK), jnp.int32),
            pltpu.VMEM((1, CHUNK), jnp.float32),
            pltpu.VMEM((1, CHUNK), jnp.float32),
            pltpu.VMEM((CHUNK, HALF), jnp.float32),
            pltpu.VMEM((CHUNK, HALF), jnp.float32),
            pltpu.VMEM((PIECE, HALF), jnp.float32),
            pltpu.VMEM((PIECE, HALF), jnp.float32),
            pltpu.VMEM((PIECE, HALF), jnp.float32),
            pltpu.VMEM((PIECE, HALF), jnp.float32),
            pltpu.VMEM((PIECE, HALF), jnp.float32),
            pltpu.VMEM_SHARED((NP, HALF), jnp.float32),
            pltpu.SemaphoreType.DMA,
            pltpu.SemaphoreType.DMA,
            pltpu.SemaphoreType.DMA,
            pltpu.SemaphoreType.DMA,
            pltpu.SemaphoreType.DMA,
            pltpu.SemaphoreType.DMA,
        ],
    )
    def lgcn_kernel(table0, blob, vals, t_out, mean_out,
                    ebuf0, ebuf1, vbuf0, vbuf1, rows0, rows1,
                    pbuf, p1, p2, p3, zbuf, acc,
                    esem0, esem1, wsem0, wsem1, gsem0, gsem1):
        c = lax.axis_index("c")
        s = lax.axis_index("s")
        half_off = c * NP
        r0 = s * ROWS_PER_TILE

        ebufs = (ebuf0, ebuf1)
        vbufs = (vbuf0, vbuf1)
        rowss = (rows0, rows1)
        esems = (esem0, esem1)
        wsems = (wsem0, wsem1)
        gsems = (gsem0, gsem1)

        zv = jnp.zeros((LANES,), jnp.float32)
        for r in range(PIECE):
            for h in range(HALF // LANES):
                zbuf[r, pl.ds(h * LANES, LANES)] = zv

        def e_start(slot, j):
            pltpu.async_copy(blob.at[s, j], ebufs[slot], esems[slot])
            pltpu.async_copy(vals.at[s, j], vbufs[slot], wsems[slot])

        def e_wait(slot, j):
            pltpu.make_async_copy(blob.at[s, j], ebufs[slot],
                                  esems[slot]).wait()
            pltpu.make_async_copy(vals.at[s, j], vbufs[slot],
                                  wsems[slot]).wait()

        def adjust(slot, gather_base):
            eb = ebufs[slot]
            for k in range(CHUNK // LANES):
                sl = pl.ds(k * LANES, LANES)
                eb[0, sl] = eb[0, sl] + gather_base

        def g_start(slot):
            pltpu.async_copy(t_out.at[ebufs[slot].at[0]], rowss[slot],
                             gsems[slot])

        def g_wait(slot):
            pltpu.make_async_copy(t_out.at[ebufs[slot].at[0]], rowss[slot],
                                  gsems[slot]).wait()

        def mul_rows(slot):
            rows = rowss[slot]
            vb = vbufs[slot]
            for g in range(CHUNK // LANES):
                vv = vb[0, pl.ds(g * LANES, LANES)]
                for e in range(LANES):
                    b = jnp.take_along_axis(
                        vv, jnp.full((LANES,), e, jnp.int32), axis=0)
                    r = g * LANES + e
                    for h in range(HALF // LANES):
                        sl = pl.ds(h * LANES, LANES)
                        rows[r, sl] = rows[r, sl] * b

        def scatter(slot):
            pltpu.sync_copy(rowss[slot], acc.at[ebufs[slot].at[1]], add=True)

        def zero_acc():
            def zbody(p, _):
                pltpu.sync_copy(zbuf, acc.at[pl.ds(r0 + p * PIECE, PIECE)])
                return _
            lax.fori_loop(0, NPIECE, zbody, 0)

        def writeback(out_base):
            def wbody(p, _):
                off = r0 + p * PIECE
                pltpu.sync_copy(acc.at[pl.ds(off, PIECE)], pbuf)
                pltpu.sync_copy(
                    pbuf, t_out.at[pl.ds(out_base + half_off + off, PIECE)])
                return _
            lax.fori_loop(0, NPIECE, wbody, 0)

        def cbody(p, _):
            off = half_off + r0 + p * PIECE
            pltpu.sync_copy(table0.at[pl.ds(off, PIECE)], pbuf)
            pltpu.sync_copy(pbuf, t_out.at[pl.ds(off, PIECE)])
            return _
        lax.fori_loop(0, NPIECE, cbody, 0)
        plsc.subcore_barrier()

        def layer(i, _):
            gather_base = half_off + i * SLAB
            out_base = (i + 1) * SLAB
            zero_acc()
            plsc.subcore_barrier()

            e_start(0, 0)
            e_wait(0, 0)
            adjust(0, gather_base)
            g_start(0)
            e_start(1, 1)

            def ebody(jj, _):
                j0 = 2 * jj
                g_wait(0)
                mul_rows(0)
                scatter(0)
                e_wait(1, j0 + 1)
                adjust(1, gather_base)
                g_start(1)
                e_start(0, j0 + 2)
                g_wait(1)
                mul_rows(1)
                scatter(1)
                e_wait(0, j0 + 2)
                adjust(0, gather_base)
                g_start(0)
                e_start(1, j0 + 3)
                return _
            lax.fori_loop(0, NCHUNK // 2, ebody, 0)
            g_wait(0)
            e_wait(1, NCHUNK + 1)

            plsc.subcore_barrier()
            writeback(out_base)
            plsc.subcore_barrier()
            return _
        lax.fori_loop(0, LAYERS, layer, 0)

        def mbody(p, _):
            hb = half_off + r0 + p * PIECE
            pltpu.sync_copy(t_out.at[pl.ds(hb, PIECE)], pbuf)
            pltpu.sync_copy(t_out.at[pl.ds(SLAB + hb, PIECE)], p1)
            pltpu.sync_copy(t_out.at[pl.ds(2 * SLAB + hb, PIECE)], p2)
            pltpu.sync_copy(t_out.at[pl.ds(3 * SLAB + hb, PIECE)], p3)

            def rbody(r, _):
                for h in range(HALF // LANES):
                    sl = pl.ds(h * LANES, LANES)
                    m = (pbuf[r, sl] + p1[r, sl] + p2[r, sl] + p3[r, sl]) \
                        * 0.25
                    pbuf[r, sl] = m
                return _
            lax.fori_loop(0, PIECE, rbody, 0)
            pltpu.sync_copy(pbuf, mean_out.at[pl.ds(hb, PIECE)])
            return _
        lax.fori_loop(0, NPIECE, mbody, 0)

    return lgcn_kernel


_LGCN = _build_kernel()


def kernel(user_emb, item_emb, adj_values, adj_indices):
    emb0 = jnp.concatenate([user_emb, item_emb], axis=0)
    rpad = NP - NODES
    table0 = jnp.concatenate(
        [jnp.pad(emb0[:, :HALF], ((0, rpad), (0, 0))),
         jnp.pad(emb0[:, HALF:], ((0, rpad), (0, 0)))], axis=0)

    row = adj_indices[0]
    col = adj_indices[1]
    pad = NS * NCHUNK * CHUNK - EDGES
    colp = jnp.pad(col, (0, pad))
    rowp = jnp.pad(row, (0, pad))
    valp = jnp.pad(adj_values, (0, pad))
    blob = jnp.stack([colp, rowp], axis=0)
    blob = blob.reshape(2, NS, NCHUNK, CHUNK).transpose(1, 2, 0, 3)
    blob = jnp.pad(blob, ((0, 0), (0, NCHUNK_ALLOC - NCHUNK), (0, 0), (0, 0)))
    vals = valp.reshape(NS, NCHUNK, 1, CHUNK)
    vals = jnp.pad(vals, ((0, 0), (0, NCHUNK_ALLOC - NCHUNK), (0, 0), (0, 0)))

    t_out, mean_out = _LGCN(table0, blob, vals)

    def unsplit(t):
        return jnp.concatenate([t[:NODES], t[NP:NP + NODES]], axis=1)

    layers = t_out.reshape(LAYERS + 1, SLAB, HALF)
    all_emb = jnp.stack([emb0] + [unsplit(layers[i + 1])
                                  for i in range(LAYERS)], axis=0)
    final = unsplit(mean_out)
    return (final[:USER_N], final[USER_N:], all_emb)


# speedup vs baseline: 6.0582x; 1.2127x over previous
import functools

import jax
import jax.numpy as jnp
from jax import lax
from jax.experimental import pallas as pl
from jax.experimental.pallas import tpu as pltpu
from jax.experimental.pallas import tpu_sc as plsc

USER_N = 25000
ITEM_N = 25000
NODES = USER_N + ITEM_N
EMB = 64
HALF = EMB // 2
LAYERS = 3
EDGES = 800000

NC = 2
NS = 16
LANES = 16
CHUNK = 128
NP = 50048
ROWS_PER_TILE = NP // NS
PIECE = 136
NPIECE = ROWS_PER_TILE // PIECE
TAIL = NODES - (NP - ROWS_PER_TILE) - (NPIECE - 1) * PIECE
_NCH = -(-EDGES // (NS * CHUNK))
NCHUNK = _NCH + (_NCH % 2)
NCHUNK_ALLOC = NCHUNK + 2
SLAB = NC * NP


def _build_kernel():
    mesh = plsc.VectorSubcoreMesh(core_axis_name="c", subcore_axis_name="s")

    @functools.partial(
        pl.kernel,
        out_type=(
            jax.ShapeDtypeStruct(((LAYERS + 1) * SLAB, HALF), jnp.float32),
            jax.ShapeDtypeStruct((LAYERS + 1, NODES, EMB), jnp.float32),
            jax.ShapeDtypeStruct((NODES, EMB), jnp.float32),
        ),
        mesh=mesh,
        compiler_params=pltpu.CompilerParams(use_tc_tiling_on_sc=False),
        scratch_types=[
            pltpu.VMEM((2, CHUNK), jnp.int32),
            pltpu.VMEM((2, CHUNK), jnp.int32),
            pltpu.VMEM((1, CHUNK), jnp.float32),
            pltpu.VMEM((1, CHUNK), jnp.float32),
            pltpu.VMEM((CHUNK, HALF), jnp.float32),
            pltpu.VMEM((CHUNK, HALF), jnp.float32),
            pltpu.VMEM((PIECE, HALF), jnp.float32),
            pltpu.VMEM((PIECE, HALF), jnp.float32),
            pltpu.VMEM((PIECE, HALF), jnp.float32),
            pltpu.VMEM((PIECE, HALF), jnp.float32),
            pltpu.VMEM((PIECE, HALF), jnp.float32),
            pltpu.VMEM_SHARED((NP, HALF), jnp.float32),
            pltpu.SemaphoreType.DMA,
            pltpu.SemaphoreType.DMA,
            pltpu.SemaphoreType.DMA,
            pltpu.SemaphoreType.DMA,
        ],
    )
    def lgcn_kernel(table0, blob, vals, t_out, all_out, mean_out,
                    ebuf0, ebuf1, vbuf0, vbuf1, rows0, rows1,
                    pbuf, p1, p2, p3, zbuf, acc,
                    esem0, esem1, gsem0, gsem1):
        c = lax.axis_index("c")
        s = lax.axis_index("s")
        half_off = c * NP
        r0 = s * ROWS_PER_TILE

        ebufs = (ebuf0, ebuf1)
        vbufs = (vbuf0, vbuf1)
        rowss = (rows0, rows1)
        esems = (esem0, esem1)
        gsems = (gsem0, gsem1)

        zv = jnp.zeros((LANES,), jnp.float32)
        for r in range(PIECE):
            for h in range(HALF // LANES):
                zbuf[r, pl.ds(h * LANES, LANES)] = zv

        def e_start(slot, j):
            pltpu.async_copy(blob.at[0, s, j], ebufs[slot].at[0], esems[slot])
            pltpu.async_copy(blob.at[1, s, j], ebufs[slot].at[1], esems[slot])
            pltpu.async_copy(vals.at[s, j], vbufs[slot], esems[slot])

        def e_wait(slot, j):
            pltpu.make_async_copy(blob.at[0, s, j], ebufs[slot].at[0],
                                  esems[slot]).wait()
            pltpu.make_async_copy(blob.at[1, s, j], ebufs[slot].at[1],
                                  esems[slot]).wait()
            pltpu.make_async_copy(vals.at[s, j], vbufs[slot],
                                  esems[slot]).wait()

        def adjust(slot, gather_base):
            eb = ebufs[slot]
            for k in range(CHUNK // LANES):
                sl = pl.ds(k * LANES, LANES)
                eb[0, sl] = eb[0, sl] + gather_base

        def g_start(slot):
            pltpu.async_copy(t_out.at[ebufs[slot].at[0]], rowss[slot],
                             gsems[slot])

        def g_wait(slot):
            pltpu.make_async_copy(t_out.at[ebufs[slot].at[0]], rowss[slot],
                                  gsems[slot]).wait()

        def mul_rows(slot):
            rows = rowss[slot]
            vb = vbufs[slot]
            for g in range(CHUNK // LANES):
                vv = vb[0, pl.ds(g * LANES, LANES)]
                for e in range(LANES):
                    b = jnp.take_along_axis(
                        vv, jnp.full((LANES,), e, jnp.int32), axis=0)
                    r = g * LANES + e
                    for h in range(HALF // LANES):
                        sl = pl.ds(h * LANES, LANES)
                        rows[r, sl] = rows[r, sl] * b

        def scatter(slot):
            pltpu.sync_copy(rowss[slot], acc.at[ebufs[slot].at[1]], add=True)

        def zero_acc():
            def zbody(p, _):
                pltpu.sync_copy(zbuf, acc.at[pl.ds(r0 + p * PIECE, PIECE)])
                return _
            lax.fori_loop(0, NPIECE, zbody, 0)

        def write_final(dst_is_mean, li, roff, src):
            full = roff <= NODES - PIECE
            for ci in range(NC):
                sel = jnp.logical_and(full, c == ci)

                @pl.when(sel)
                def _():
                    cs = pl.ds(ci * HALF, HALF)
                    if dst_is_mean:
                        dst = mean_out.at[pl.ds(roff, PIECE), cs]
                    else:
                        dst = all_out.at[li, pl.ds(roff, PIECE), cs]
                    pltpu.sync_copy(src, dst)

                selt = jnp.logical_and(jnp.logical_not(full), c == ci)

                @pl.when(selt)
                def _():
                    cs = pl.ds(ci * HALF, HALF)
                    if dst_is_mean:
                        dst = mean_out.at[pl.ds(roff, TAIL), cs]
                    else:
                        dst = all_out.at[li, pl.ds(roff, TAIL), cs]
                    pltpu.sync_copy(src.at[pl.ds(0, TAIL)], dst)

        def writeback(li, out_base):
            def wbody(p, _):
                off = r0 + p * PIECE
                pltpu.sync_copy(acc.at[pl.ds(off, PIECE)], pbuf)
                pltpu.sync_copy(
                    pbuf, t_out.at[pl.ds(out_base + half_off + off, PIECE)])
                write_final(False, li, off, pbuf)
                return _
            lax.fori_loop(0, NPIECE, wbody, 0)

        def cbody(p, _):
            off = r0 + p * PIECE
            pltpu.sync_copy(table0.at[pl.ds(half_off + off, PIECE)], pbuf)
            pltpu.sync_copy(pbuf, t_out.at[pl.ds(half_off + off, PIECE)])
            write_final(False, 0, off, pbuf)
            return _
        lax.fori_loop(0, NPIECE, cbody, 0)
        plsc.subcore_barrier()

        def layer(i, _):
            gather_base = half_off + i * SLAB
            out_base = (i + 1) * SLAB
            zero_acc()
            plsc.subcore_barrier()

            e_start(0, 0)
            e_wait(0, 0)
            adjust(0, gather_base)
            g_start(0)
            e_start(1, 1)

            def ebody(jj, _):
                j0 = 2 * jj
                g_wait(0)
                mul_rows(0)
                scatter(0)
                e_wait(1, j0 + 1)
                adjust(1, gather_base)
                g_start(1)
                e_start(0, j0 + 2)
                g_wait(1)
                mul_rows(1)
                scatter(1)
                e_wait(0, j0 + 2)
                adjust(0, gather_base)
                g_start(0)
                e_start(1, j0 + 3)
                return _
            lax.fori_loop(0, NCHUNK // 2, ebody, 0)
            g_wait(0)
            e_wait(1, NCHUNK + 1)

            plsc.subcore_barrier()
            writeback(i + 1, out_base)
            plsc.subcore_barrier()
            return _
        lax.fori_loop(0, LAYERS, layer, 0)

        def mbody(p, _):
            off = r0 + p * PIECE
            hb = half_off + off
            pltpu.sync_copy(t_out.at[pl.ds(hb, PIECE)], pbuf)
            pltpu.sync_copy(t_out.at[pl.ds(SLAB + hb, PIECE)], p1)
            pltpu.sync_copy(t_out.at[pl.ds(2 * SLAB + hb, PIECE)], p2)
            pltpu.sync_copy(t_out.at[pl.ds(3 * SLAB + hb, PIECE)], p3)

            def rbody(r, _):
                for h in range(HALF // LANES):
                    sl = pl.ds(h * LANES, LANES)
                    m = (pbuf[r, sl] + p1[r, sl] + p2[r, sl] + p3[r, sl]) \
                        * 0.25
                    pbuf[r, sl] = m
                return _
            lax.fori_loop(0, PIECE, rbody, 0)
            write_final(True, 0, off, pbuf)
            return _
        lax.fori_loop(0, NPIECE, mbody, 0)

    return lgcn_kernel


_LGCN = _build_kernel()


def kernel(user_emb, item_emb, adj_values, adj_indices):
    emb0 = jnp.concatenate([user_emb, item_emb], axis=0)
    rpad = NP - NODES
    table0 = jnp.concatenate(
        [jnp.pad(emb0[:, :HALF], ((0, rpad), (0, 0))),
         jnp.pad(emb0[:, HALF:], ((0, rpad), (0, 0)))], axis=0)

    row = adj_indices[0]
    col = adj_indices[1]
    pad = NS * NCHUNK * CHUNK - EDGES
    colp = jnp.pad(col, (0, pad))
    rowp = jnp.pad(row, (0, pad))
    valp = jnp.pad(adj_values, (0, pad))
    blob = jnp.stack([colp, rowp], axis=0).reshape(2, NS, NCHUNK, CHUNK)
    blob = jnp.pad(blob, ((0, 0), (0, 0), (0, NCHUNK_ALLOC - NCHUNK), (0, 0)))
    vals = valp.reshape(NS, NCHUNK, 1, CHUNK)
    vals = jnp.pad(vals, ((0, 0), (0, NCHUNK_ALLOC - NCHUNK), (0, 0), (0, 0)))

    _, all_emb, final = _LGCN(table0, blob, vals)
    return (final[:USER_N], final[USER_N:], all_emb)
